# baseline (speedup 1.0000x reference)
import functools

import jax
import jax.numpy as jnp
from jax import lax
from jax.experimental import pallas as pl
from jax.experimental.pallas import tpu as pltpu
from jax.experimental.pallas import tpu_sc as plsc

B = 4096
SEQ = 200
D = 64
C = 30
CHUNK = 100
NCHUNK = SEQ // CHUNK
NLANE = 16
NVREG = D // NLANE


def _make_sc_pool():
    info = plsc.get_sparse_core_info()
    nc, ns = info.num_cores, info.num_subcores
    nw = nc * ns
    tpw = B // nw

    mesh = plsc.VectorSubcoreMesh(core_axis_name="c", subcore_axis_name="s")

    @functools.partial(
        pl.kernel,
        out_type=jax.ShapeDtypeStruct((B, D), jnp.float32),
        mesh=mesh,
        compiler_params=pltpu.CompilerParams(use_tc_tiling_on_sc=False),
        scratch_types=[
            pltpu.VMEM((tpw * NCHUNK, CHUNK), jnp.int32),
            pltpu.VMEM((2, SEQ, D), jnp.float32),
            pltpu.VMEM((tpw, D), jnp.float32),
            pltpu.SemaphoreType.DMA,
        ],
    )
    def pool(idx_hbm, table_hbm, out_hbm, idx_v, rows_v, out_v, sem):
        wid = lax.axis_index("s") * nc + lax.axis_index("c")
        base = wid * tpw
        pltpu.sync_copy(idx_hbm.at[pl.ds(base * NCHUNK, tpw * NCHUNK)], idx_v)

        def fire(t, buf):
            for j in range(NCHUNK):
                pltpu.async_copy(
                    table_hbm.at[idx_v.at[t * NCHUNK + j]],
                    rows_v.at[buf, pl.ds(j * CHUNK, CHUNK)],
                    sem,
                )

        def drain(buf):
            for j in range(NCHUNK):
                pltpu.make_async_copy(
                    table_hbm.at[idx_v.at[j]],
                    rows_v.at[buf, pl.ds(j * CHUNK, CHUNK)],
                    sem,
                ).wait()

        fire(0, 0)

        def text_body(t, carry):
            buf = lax.rem(t, 2)
            drain(buf)
            @pl.when(t + 1 < tpw)
            def _():
                fire(t + 1, 1 - buf)

            def row_body(r, accs):
                return tuple(
                    accs[k] + rows_v[buf, r, pl.ds(NLANE * k, NLANE)]
                    for k in range(NVREG)
                )

            accs = lax.fori_loop(
                0, SEQ, row_body,
                tuple(jnp.zeros((NLANE,), jnp.float32) for _ in range(NVREG)),
            )
            for k in range(NVREG):
                out_v[t, pl.ds(NLANE * k, NLANE)] = accs[k]
            return carry

        lax.fori_loop(0, tpw, text_body, 0)
        pltpu.sync_copy(out_v, out_hbm.at[pl.ds(base, tpw)])

    return pool


def _classifier_body(x_ref, w_ref, b_ref, probs_ref, pred_ref):
    x = x_ref[...] / jnp.float32(SEQ)
    logits = lax.dot_general(
        x, w_ref[...], (((1,), (1,)), ((), ())),
        preferred_element_type=jnp.float32,
    ) + b_ref[...]
    m = jnp.max(logits, axis=1, keepdims=True)
    e = jnp.exp(logits - m)
    p = e / jnp.sum(e, axis=1, keepdims=True)
    probs_ref[...] = p
    pm = jnp.max(p, axis=1, keepdims=True)
    iota = lax.broadcasted_iota(jnp.int32, p.shape, 1)
    pred_ref[...] = jnp.min(jnp.where(p == pm, iota, C), axis=1, keepdims=True)


def _tc_classifier(xsum, w, b2):
    return pl.pallas_call(
        _classifier_body,
        out_shape=(
            jax.ShapeDtypeStruct((B, C), jnp.float32),
            jax.ShapeDtypeStruct((B, 1), jnp.int32),
        ),
    )(xsum, w, b2)


def kernel(indices, table, W, b):
    idx2 = indices.reshape(B * NCHUNK, CHUNK).astype(jnp.int32)
    xsum = _make_sc_pool()(idx2, table)
    probs, pred = _tc_classifier(xsum, W, b.reshape(1, C))
    return (pred.reshape(B), probs)
